# Initial kernel scaffold; baseline (speedup 1.0000x reference)
import jax
import jax.numpy as jnp
from jax.experimental import pallas as pl


def kernel(h, edge_index, edge_type, num_nodes, W_msg, b_msg, edge_embed, W_ih, W_hh, b_ih, b_hh):
    raise NotImplementedError("write your pallas kernel here")


# speedup vs baseline: 5.7296x; 5.7296x over previous
import functools

import jax
import jax.numpy as jnp
from jax import lax
from jax.experimental import pallas as pl
from jax.experimental.pallas import tpu as pltpu
from jax.experimental.pallas import tpu_sc as plsc


def _make_sc_edge_accum(N, D, G, rows_per_tile):
    info = plsc.get_sparse_core_info()
    NC, NS = info.num_cores, info.num_subcores
    Dh = D // 2
    mesh = plsc.VectorSubcoreMesh(core_axis_name="c", subcore_axis_name="s")
    nrt = (N // NS) // 8 * 8
    tail = N - nrt * NS

    @functools.partial(
        pl.kernel,
        out_type=(
            jax.ShapeDtypeStruct((NC, N, Dh), jnp.float32),
            jax.ShapeDtypeStruct((N, 16), jnp.float32),
        ),
        mesh=mesh,
        scratch_types=[
            pltpu.VMEM_SHARED((N, Dh), jnp.float32),
            pltpu.VMEM_SHARED((N, 16), jnp.float32),
            pltpu.VMEM((rows_per_tile, G), jnp.int32),
            pltpu.VMEM((rows_per_tile, G), jnp.int32),
            pltpu.VMEM((rows_per_tile, G), jnp.int32),
            pltpu.VMEM((G, Dh), jnp.float32),
            pltpu.VMEM((G, 16), jnp.float32),
            pltpu.SemaphoreType.DMA,
        ],
        compiler_params=pltpu.CompilerParams(use_tc_tiling_on_sc=False),
    )
    def sc_fn(h2_hbm, srcA_hbm, srcB_hbm, dst_hbm, et_hbm, zacc_hbm, zcnt_hbm,
              acc_out, cnt_out,
              acc_sh, cnt_sh, src_v, dst_v, et_v, rows_v, oh_v, sem_h):
        c = lax.axis_index("c")
        s = lax.axis_index("s")

        ra = s * nrt
        pltpu.sync_copy(zacc_hbm.at[pl.ds(ra, nrt)], acc_sh.at[pl.ds(ra, nrt)])

        @pl.when(s == 0)
        def _():
            if tail:
                t0 = nrt * NS
                pltpu.sync_copy(zacc_hbm.at[pl.ds(t0, tail)],
                                acc_sh.at[pl.ds(t0, tail)])

        @pl.when(c == 0)
        def _():
            pltpu.sync_copy(zcnt_hbm.at[pl.ds(ra, nrt)],
                            cnt_sh.at[pl.ds(ra, nrt)])

            @pl.when(s == 0)
            def _():
                if tail:
                    t0 = nrt * NS
                    pltpu.sync_copy(zcnt_hbm.at[pl.ds(t0, tail)],
                                    cnt_sh.at[pl.ds(t0, tail)])

        g0 = s * rows_per_tile

        @pl.when(c == 0)
        def _():
            pltpu.sync_copy(srcA_hbm.at[pl.ds(g0, rows_per_tile)], src_v)

        @pl.when(c == 1)
        def _():
            pltpu.sync_copy(srcB_hbm.at[pl.ds(g0, rows_per_tile)], src_v)

        pltpu.sync_copy(dst_hbm.at[pl.ds(g0, rows_per_tile)], dst_v)

        @pl.when(c == 0)
        def _():
            pltpu.sync_copy(et_hbm.at[pl.ds(g0, rows_per_tile)], et_v)

        plsc.subcore_barrier()

        lanes = lax.broadcasted_iota(jnp.int32, (16,), 0)

        def body(k, carry):
            cp_h = pltpu.async_copy(h2_hbm.at[src_v.at[k]], rows_v, sem_h)

            @pl.when(c == 0)
            def _():
                for jj in range(G // 16):
                    etv = et_v[k, pl.ds(16 * jj, 16)]
                    for i in range(16):
                        oh_v[16 * jj + i] = jnp.where(lanes == etv[i],
                                                      1.0, 0.0)

            cp_h.wait()
            pltpu.sync_copy(rows_v, acc_sh.at[dst_v.at[k]], add=True)

            @pl.when(c == 0)
            def _():
                pltpu.sync_copy(oh_v, cnt_sh.at[dst_v.at[k]], add=True)

            return carry

        lax.fori_loop(0, rows_per_tile, body, 0)

        plsc.subcore_barrier()
        pltpu.sync_copy(acc_sh.at[pl.ds(ra, nrt)],
                        acc_out.at[c, pl.ds(ra, nrt)])

        @pl.when(s == 0)
        def _():
            if tail:
                t0 = nrt * NS
                pltpu.sync_copy(acc_sh.at[pl.ds(t0, tail)],
                                acc_out.at[c, pl.ds(t0, tail)])

        @pl.when(c == 0)
        def _():
            pltpu.sync_copy(cnt_sh.at[pl.ds(ra, nrt)],
                            cnt_out.at[pl.ds(ra, nrt)])

            @pl.when(s == 0)
            def _():
                if tail:
                    t0 = nrt * NS
                    pltpu.sync_copy(cnt_sh.at[pl.ds(t0, tail)],
                                    cnt_out.at[pl.ds(t0, tail)])

    return sc_fn


def _make_tc_gru(N, D, R, T1):
    T3 = 3 * D
    Dh = D // 2
    hi = lax.Precision.HIGHEST

    def body(nn_ref, acc_ref, cnt_ref, h_ref, wm_ref, epad_ref, bmsg_ref,
             wih_ref, whh_ref, bih_ref, bhh_ref, out_ref):
        hh = h_ref[...]
        trow = lax.broadcasted_iota(jnp.int32, (16, D), 0)
        embp = epad_ref[...] + jnp.where(trow < T1, bmsg_ref[...], 0.0)
        dn = (((1,), (1,)), ((), ()))
        msgs = (
            lax.dot_general(acc_ref[0], wm_ref[:, :Dh], dn, precision=hi,
                            preferred_element_type=jnp.float32)
            + lax.dot_general(acc_ref[1], wm_ref[:, Dh:], dn, precision=hi,
                              preferred_element_type=jnp.float32)
            + jnp.dot(cnt_ref[...], embp, precision=hi,
                      preferred_element_type=jnp.float32)
        )
        gi = lax.dot_general(msgs, wih_ref[...], dn, precision=hi,
                             preferred_element_type=jnp.float32) + bih_ref[...]
        gh = lax.dot_general(hh, whh_ref[...], dn, precision=hi,
                             preferred_element_type=jnp.float32) + bhh_ref[...]
        r = jax.nn.sigmoid(gi[:, :D] + gh[:, :D])
        z = jax.nn.sigmoid(gi[:, D:2 * D] + gh[:, D:2 * D])
        n = jnp.tanh(gi[:, 2 * D:] + r * gh[:, 2 * D:])
        h_new = (1.0 - z) * n + z * hh
        row0 = pl.program_id(0) * R
        rows = row0 + lax.broadcasted_iota(jnp.int32, (R, D), 0)
        out_ref[...] = jnp.where(rows < nn_ref[0], h_new, 0.0)

    return pl.pallas_call(
        body,
        grid=(N // R,),
        in_specs=[
            pl.BlockSpec(memory_space=pltpu.MemorySpace.SMEM),
            pl.BlockSpec((2, R, Dh), lambda j: (0, j, 0)),
            pl.BlockSpec((R, 16), lambda j: (j, 0)),
            pl.BlockSpec((R, D), lambda j: (j, 0)),
            pl.BlockSpec((D, D), lambda j: (0, 0)),
            pl.BlockSpec((16, D), lambda j: (0, 0)),
            pl.BlockSpec((1, D), lambda j: (0, 0)),
            pl.BlockSpec((T3, D), lambda j: (0, 0)),
            pl.BlockSpec((T3, D), lambda j: (0, 0)),
            pl.BlockSpec((1, T3), lambda j: (0, 0)),
            pl.BlockSpec((1, T3), lambda j: (0, 0)),
        ],
        out_specs=pl.BlockSpec((R, D), lambda j: (j, 0)),
        out_shape=jax.ShapeDtypeStruct((N, D), jnp.float32),
    )


def kernel(h, edge_index, edge_type, num_nodes, W_msg, b_msg, edge_embed,
           W_ih, W_hh, b_ih, b_hh):
    N, D = h.shape
    E = edge_index.shape[1]
    T1 = edge_embed.shape[0]

    G = 80
    n_groups = E // G
    NS = 16
    rows_per_tile = n_groups // NS

    src = edge_index[0].astype(jnp.int32)
    srcA = (2 * src).reshape(n_groups, G)
    srcB = (2 * src + 1).reshape(n_groups, G)
    dst = edge_index[1].astype(jnp.int32).reshape(n_groups, G)
    et = jnp.clip(edge_type, 0, T1 - 1).astype(jnp.int32).reshape(n_groups, G)
    h2 = h.reshape(2 * N, D // 2)
    zacc = jnp.zeros((N, D // 2), jnp.float32)
    zcnt = jnp.zeros((N, 16), jnp.float32)

    sc_fn = _make_sc_edge_accum(N, D, G, rows_per_tile)
    acc2, cnt = sc_fn(h2, srcA, srcB, dst, et, zacc, zcnt)

    e_pad = jnp.zeros((16, D), jnp.float32).at[:T1].set(edge_embed)
    nn = jnp.asarray(num_nodes, jnp.int32).reshape(1)
    tc_fn = _make_tc_gru(N, D, 1000, T1)
    return tc_fn(nn, acc2, cnt, h, W_msg, e_pad, b_msg.reshape(1, D),
                 W_ih, W_hh, b_ih.reshape(1, 3 * D), b_hh.reshape(1, 3 * D))


# speedup vs baseline: 8.1911x; 1.4296x over previous
import functools

import jax
import jax.numpy as jnp
from jax import lax
from jax.experimental import pallas as pl
from jax.experimental.pallas import tpu as pltpu
from jax.experimental.pallas import tpu_sc as plsc


def _make_sc_edge_accum(N, D, G, rows_per_tile):
    info = plsc.get_sparse_core_info()
    NC, NS = info.num_cores, info.num_subcores
    Dh = D // 2
    mesh = plsc.VectorSubcoreMesh(core_axis_name="c", subcore_axis_name="s")
    nrt = (N // NS) // 8 * 8
    tail = N - nrt * NS

    @functools.partial(
        pl.kernel,
        out_type=(
            jax.ShapeDtypeStruct((NC, N, Dh), jnp.float32),
            jax.ShapeDtypeStruct((NC, N, 16), jnp.float32),
        ),
        mesh=mesh,
        scratch_types=[
            pltpu.VMEM_SHARED((N, Dh), jnp.float32),
            pltpu.VMEM_SHARED((N, 16), jnp.float32),
            pltpu.VMEM((rows_per_tile, G), jnp.int32),
            pltpu.VMEM((rows_per_tile, G), jnp.int32),
            pltpu.VMEM((rows_per_tile, G), jnp.int32),
            pltpu.VMEM((2, G, Dh), jnp.float32),
            pltpu.VMEM((G, 16), jnp.float32),
            pltpu.SemaphoreType.DMA,
            pltpu.SemaphoreType.DMA,
        ],
        compiler_params=pltpu.CompilerParams(use_tc_tiling_on_sc=False),
    )
    def sc_fn(h2_hbm, srcA_hbm, srcB_hbm, dst_hbm, et_hbm, zacc_hbm, zcnt_hbm,
              acc_out, cnt_out,
              acc_sh, cnt_sh, src_v, dst_v, et_v, rows_v, oh_v,
              sem_g0, sem_g1):
        c = lax.axis_index("c")
        s = lax.axis_index("s")

        ra = s * nrt
        pltpu.sync_copy(zacc_hbm.at[pl.ds(ra, nrt)], acc_sh.at[pl.ds(ra, nrt)])

        @pl.when(s == 0)
        def _():
            if tail:
                t0 = nrt * NS
                pltpu.sync_copy(zacc_hbm.at[pl.ds(t0, tail)],
                                acc_sh.at[pl.ds(t0, tail)])

        pltpu.sync_copy(zcnt_hbm.at[pl.ds(ra, nrt)],
                        cnt_sh.at[pl.ds(ra, nrt)])

        @pl.when(s == 0)
        def _():
            if tail:
                t0 = nrt * NS
                pltpu.sync_copy(zcnt_hbm.at[pl.ds(t0, tail)],
                                cnt_sh.at[pl.ds(t0, tail)])

        g0 = s * rows_per_tile

        @pl.when(c == 0)
        def _():
            pltpu.sync_copy(srcA_hbm.at[pl.ds(g0, rows_per_tile)], src_v)

        @pl.when(c == 1)
        def _():
            pltpu.sync_copy(srcB_hbm.at[pl.ds(g0, rows_per_tile)], src_v)

        pltpu.sync_copy(dst_hbm.at[pl.ds(g0, rows_per_tile)], dst_v)

        pltpu.sync_copy(et_hbm.at[pl.ds(g0, rows_per_tile)], et_v)

        plsc.subcore_barrier()

        lanes = lax.broadcasted_iota(jnp.int32, (16,), 0)
        sems = (sem_g0, sem_g1)
        n = rows_per_tile

        pltpu.async_copy(h2_hbm.at[src_v.at[0]], rows_v.at[0], sem_g0)

        def body(k2, carry):
            k = 2 * k2
            for b in range(2):
                g = k + b
                ob = 1 - b
                if b == 0:
                    pltpu.async_copy(h2_hbm.at[src_v.at[g + 1]],
                                     rows_v.at[ob], sems[ob])
                else:
                    @pl.when(g + 1 < n)
                    def _():
                        pltpu.async_copy(h2_hbm.at[src_v.at[g + 1]],
                                         rows_v.at[ob], sems[ob])

                @pl.when(c == b)
                def _():
                    for jj in range(G // 16):
                        etv = et_v[g, pl.ds(16 * jj, 16)]
                        for i in range(16):
                            oh_v[16 * jj + i] = jnp.where(lanes == etv[i],
                                                          1.0, 0.0)

                pltpu.make_async_copy(h2_hbm.at[src_v.at[g]],
                                      rows_v.at[b], sems[b]).wait()
                pltpu.sync_copy(rows_v.at[b], acc_sh.at[dst_v.at[g]], add=True)

                @pl.when(c == b)
                def _():
                    pltpu.sync_copy(oh_v, cnt_sh.at[dst_v.at[g]], add=True)

            return carry

        lax.fori_loop(0, n // 2, body, 0)

        plsc.subcore_barrier()
        pltpu.sync_copy(acc_sh.at[pl.ds(ra, nrt)],
                        acc_out.at[c, pl.ds(ra, nrt)])

        @pl.when(s == 0)
        def _():
            if tail:
                t0 = nrt * NS
                pltpu.sync_copy(acc_sh.at[pl.ds(t0, tail)],
                                acc_out.at[c, pl.ds(t0, tail)])

        pltpu.sync_copy(cnt_sh.at[pl.ds(ra, nrt)],
                        cnt_out.at[c, pl.ds(ra, nrt)])

        @pl.when(s == 0)
        def _():
            if tail:
                t0 = nrt * NS
                pltpu.sync_copy(cnt_sh.at[pl.ds(t0, tail)],
                                cnt_out.at[c, pl.ds(t0, tail)])

    return sc_fn


def _make_tc_gru(N, D, R, T1):
    T3 = 3 * D
    Dh = D // 2
    hi = lax.Precision.HIGHEST

    def body(nn_ref, acc_ref, cnt_ref, h_ref, wm_ref, epad_ref, bmsg_ref,
             wih_ref, whh_ref, bih_ref, bhh_ref, out_ref):
        hh = h_ref[...]
        trow = lax.broadcasted_iota(jnp.int32, (16, D), 0)
        embp = epad_ref[...] + jnp.where(trow < T1, bmsg_ref[...], 0.0)
        dn = (((1,), (1,)), ((), ()))
        msgs = (
            lax.dot_general(acc_ref[0], wm_ref[:, :Dh], dn, precision=hi,
                            preferred_element_type=jnp.float32)
            + lax.dot_general(acc_ref[1], wm_ref[:, Dh:], dn, precision=hi,
                              preferred_element_type=jnp.float32)
            + jnp.dot(cnt_ref[0] + cnt_ref[1], embp, precision=hi,
                      preferred_element_type=jnp.float32)
        )
        gi = lax.dot_general(msgs, wih_ref[...], dn, precision=hi,
                             preferred_element_type=jnp.float32) + bih_ref[...]
        gh = lax.dot_general(hh, whh_ref[...], dn, precision=hi,
                             preferred_element_type=jnp.float32) + bhh_ref[...]
        r = jax.nn.sigmoid(gi[:, :D] + gh[:, :D])
        z = jax.nn.sigmoid(gi[:, D:2 * D] + gh[:, D:2 * D])
        n = jnp.tanh(gi[:, 2 * D:] + r * gh[:, 2 * D:])
        h_new = (1.0 - z) * n + z * hh
        row0 = pl.program_id(0) * R
        rows = row0 + lax.broadcasted_iota(jnp.int32, (R, D), 0)
        out_ref[...] = jnp.where(rows < nn_ref[0], h_new, 0.0)

    return pl.pallas_call(
        body,
        grid=(N // R,),
        in_specs=[
            pl.BlockSpec(memory_space=pltpu.MemorySpace.SMEM),
            pl.BlockSpec((2, R, Dh), lambda j: (0, j, 0)),
            pl.BlockSpec((2, R, 16), lambda j: (0, j, 0)),
            pl.BlockSpec((R, D), lambda j: (j, 0)),
            pl.BlockSpec((D, D), lambda j: (0, 0)),
            pl.BlockSpec((16, D), lambda j: (0, 0)),
            pl.BlockSpec((1, D), lambda j: (0, 0)),
            pl.BlockSpec((T3, D), lambda j: (0, 0)),
            pl.BlockSpec((T3, D), lambda j: (0, 0)),
            pl.BlockSpec((1, T3), lambda j: (0, 0)),
            pl.BlockSpec((1, T3), lambda j: (0, 0)),
        ],
        out_specs=pl.BlockSpec((R, D), lambda j: (j, 0)),
        out_shape=jax.ShapeDtypeStruct((N, D), jnp.float32),
    )


def kernel(h, edge_index, edge_type, num_nodes, W_msg, b_msg, edge_embed,
           W_ih, W_hh, b_ih, b_hh):
    N, D = h.shape
    E = edge_index.shape[1]
    T1 = edge_embed.shape[0]

    G = 80
    n_groups = E // G
    NS = 16
    rows_per_tile = n_groups // NS

    src = edge_index[0].astype(jnp.int32)
    srcA = (2 * src).reshape(n_groups, G)
    srcB = (2 * src + 1).reshape(n_groups, G)
    dst = edge_index[1].astype(jnp.int32).reshape(n_groups, G)
    et = jnp.clip(edge_type, 0, T1 - 1).astype(jnp.int32).reshape(n_groups, G)
    h2 = h.reshape(2 * N, D // 2)
    zacc = jnp.zeros((N, D // 2), jnp.float32)
    zcnt = jnp.zeros((N, 16), jnp.float32)

    sc_fn = _make_sc_edge_accum(N, D, G, rows_per_tile)
    acc2, cnt = sc_fn(h2, srcA, srcB, dst, et, zacc, zcnt)

    e_pad = jnp.zeros((16, D), jnp.float32).at[:T1].set(edge_embed)
    nn = jnp.asarray(num_nodes, jnp.int32).reshape(1)
    tc_fn = _make_tc_gru(N, D, 1000, T1)
    return tc_fn(nn, acc2, cnt, h, W_msg, e_pad, b_msg.reshape(1, D),
                 W_ih, W_hh, b_ih.reshape(1, 3 * D), b_hh.reshape(1, 3 * D))


# speedup vs baseline: 10.5647x; 1.2898x over previous
import functools

import jax
import jax.numpy as jnp
from jax import lax
from jax.experimental import pallas as pl
from jax.experimental.pallas import tpu as pltpu
from jax.experimental.pallas import tpu_sc as plsc


def _make_sc_edge_accum(N, D, G, rows_per_tile):
    info = plsc.get_sparse_core_info()
    NC, NS = info.num_cores, info.num_subcores
    Dh = D // 2
    mesh = plsc.VectorSubcoreMesh(core_axis_name="c", subcore_axis_name="s")
    nrt = (N // NS) // 8 * 8
    tail = N - nrt * NS

    @functools.partial(
        pl.kernel,
        out_type=(
            jax.ShapeDtypeStruct((NC, N, Dh), jnp.float32),
            jax.ShapeDtypeStruct((NC, N, 16), jnp.float32),
        ),
        mesh=mesh,
        scratch_types=[
            pltpu.VMEM_SHARED((N, Dh), jnp.float32),
            pltpu.VMEM_SHARED((N, 16), jnp.float32),
            pltpu.VMEM((rows_per_tile, G), jnp.int32),
            pltpu.VMEM((rows_per_tile, G), jnp.int32),
            pltpu.VMEM((rows_per_tile, G), jnp.int32),
            pltpu.VMEM((2, G, Dh), jnp.float32),
            pltpu.VMEM((G, 16), jnp.float32),
            pltpu.SemaphoreType.DMA,
            pltpu.SemaphoreType.DMA,
        ],
        compiler_params=pltpu.CompilerParams(use_tc_tiling_on_sc=False),
    )
    def sc_fn(h2_hbm, srcA_hbm, srcB_hbm, dst_hbm, et_hbm, zacc_hbm, zcnt_hbm,
              acc_out, cnt_out,
              acc_sh, cnt_sh, src_v, dst_v, et_v, rows_v, oh_v,
              sem_g0, sem_g1):
        c = lax.axis_index("c")
        s = lax.axis_index("s")

        ra = s * nrt
        pltpu.sync_copy(zacc_hbm.at[pl.ds(ra, nrt)], acc_sh.at[pl.ds(ra, nrt)])

        @pl.when(s == 0)
        def _():
            if tail:
                t0 = nrt * NS
                pltpu.sync_copy(zacc_hbm.at[pl.ds(t0, tail)],
                                acc_sh.at[pl.ds(t0, tail)])

        pltpu.sync_copy(zcnt_hbm.at[pl.ds(ra, nrt)],
                        cnt_sh.at[pl.ds(ra, nrt)])

        @pl.when(s == 0)
        def _():
            if tail:
                t0 = nrt * NS
                pltpu.sync_copy(zcnt_hbm.at[pl.ds(t0, tail)],
                                cnt_sh.at[pl.ds(t0, tail)])

        g0 = s * rows_per_tile

        @pl.when(c == 0)
        def _():
            pltpu.sync_copy(srcA_hbm.at[pl.ds(g0, rows_per_tile)], src_v)

        @pl.when(c == 1)
        def _():
            pltpu.sync_copy(srcB_hbm.at[pl.ds(g0, rows_per_tile)], src_v)

        pltpu.sync_copy(dst_hbm.at[pl.ds(g0, rows_per_tile)], dst_v)

        pltpu.sync_copy(et_hbm.at[pl.ds(g0, rows_per_tile)], et_v)

        plsc.subcore_barrier()

        lanes = lax.broadcasted_iota(jnp.int32, (16,), 0)
        sems = (sem_g0, sem_g1)
        n = rows_per_tile

        pltpu.async_copy(h2_hbm.at[src_v.at[0]], rows_v.at[0], sem_g0)

        def body(k2, carry):
            k = 2 * k2
            for b in range(2):
                g = k + b
                ob = 1 - b
                if b == 0:
                    pltpu.async_copy(h2_hbm.at[src_v.at[g + 1]],
                                     rows_v.at[ob], sems[ob])
                else:
                    @pl.when(g + 1 < n)
                    def _():
                        pltpu.async_copy(h2_hbm.at[src_v.at[g + 1]],
                                         rows_v.at[ob], sems[ob])

                @pl.when(c == b)
                def _():
                    for jj in range(G // 16):
                        etv = et_v[g, pl.ds(16 * jj, 16)]
                        for i in range(16):
                            oh_v[16 * jj + i] = jnp.where(lanes == etv[i],
                                                          1.0, 0.0)

                pltpu.make_async_copy(h2_hbm.at[src_v.at[g]],
                                      rows_v.at[b], sems[b]).wait()
                pltpu.sync_copy(rows_v.at[b], acc_sh.at[dst_v.at[g]], add=True)

                @pl.when(c == b)
                def _():
                    pltpu.sync_copy(oh_v, cnt_sh.at[dst_v.at[g]], add=True)

            return carry

        lax.fori_loop(0, n // 2, body, 0)

        plsc.subcore_barrier()
        pltpu.sync_copy(acc_sh.at[pl.ds(ra, nrt)],
                        acc_out.at[c, pl.ds(ra, nrt)])

        @pl.when(s == 0)
        def _():
            if tail:
                t0 = nrt * NS
                pltpu.sync_copy(acc_sh.at[pl.ds(t0, tail)],
                                acc_out.at[c, pl.ds(t0, tail)])

        pltpu.sync_copy(cnt_sh.at[pl.ds(ra, nrt)],
                        cnt_out.at[c, pl.ds(ra, nrt)])

        @pl.when(s == 0)
        def _():
            if tail:
                t0 = nrt * NS
                pltpu.sync_copy(cnt_sh.at[pl.ds(t0, tail)],
                                cnt_out.at[c, pl.ds(t0, tail)])

    return sc_fn


def _make_tc_gru(N, D, R, T1):
    T3 = 3 * D
    Dh = D // 2
    hi = lax.Precision.DEFAULT

    def body(nn_ref, acc_ref, cnt_ref, h_ref, wm_ref, epad_ref, bmsg_ref,
             wih_ref, whh_ref, bih_ref, bhh_ref, out_ref):
        hh = h_ref[...]
        trow = lax.broadcasted_iota(jnp.int32, (16, D), 0)
        embp = epad_ref[...] + jnp.where(trow < T1, bmsg_ref[...], 0.0)
        dn = (((1,), (1,)), ((), ()))
        msgs = (
            lax.dot_general(acc_ref[0], wm_ref[:, :Dh], dn, precision=hi,
                            preferred_element_type=jnp.float32)
            + lax.dot_general(acc_ref[1], wm_ref[:, Dh:], dn, precision=hi,
                              preferred_element_type=jnp.float32)
            + jnp.dot(cnt_ref[0] + cnt_ref[1], embp, precision=hi,
                      preferred_element_type=jnp.float32)
        )
        gi = lax.dot_general(msgs, wih_ref[...], dn, precision=hi,
                             preferred_element_type=jnp.float32) + bih_ref[...]
        gh = lax.dot_general(hh, whh_ref[...], dn, precision=hi,
                             preferred_element_type=jnp.float32) + bhh_ref[...]
        r = jax.nn.sigmoid(gi[:, :D] + gh[:, :D])
        z = jax.nn.sigmoid(gi[:, D:2 * D] + gh[:, D:2 * D])
        n = jnp.tanh(gi[:, 2 * D:] + r * gh[:, 2 * D:])
        h_new = (1.0 - z) * n + z * hh
        row0 = pl.program_id(0) * R
        rows = row0 + lax.broadcasted_iota(jnp.int32, (R, D), 0)
        out_ref[...] = jnp.where(rows < nn_ref[0], h_new, 0.0)

    return pl.pallas_call(
        body,
        grid=(N // R,),
        in_specs=[
            pl.BlockSpec(memory_space=pltpu.MemorySpace.SMEM),
            pl.BlockSpec((2, R, Dh), lambda j: (0, j, 0)),
            pl.BlockSpec((2, R, 16), lambda j: (0, j, 0)),
            pl.BlockSpec((R, D), lambda j: (j, 0)),
            pl.BlockSpec((D, D), lambda j: (0, 0)),
            pl.BlockSpec((16, D), lambda j: (0, 0)),
            pl.BlockSpec((1, D), lambda j: (0, 0)),
            pl.BlockSpec((T3, D), lambda j: (0, 0)),
            pl.BlockSpec((T3, D), lambda j: (0, 0)),
            pl.BlockSpec((1, T3), lambda j: (0, 0)),
            pl.BlockSpec((1, T3), lambda j: (0, 0)),
        ],
        out_specs=pl.BlockSpec((R, D), lambda j: (j, 0)),
        out_shape=jax.ShapeDtypeStruct((N, D), jnp.float32),
    )


def kernel(h, edge_index, edge_type, num_nodes, W_msg, b_msg, edge_embed,
           W_ih, W_hh, b_ih, b_hh):
    N, D = h.shape
    E = edge_index.shape[1]
    T1 = edge_embed.shape[0]

    G = 80
    n_groups = E // G
    NS = 16
    rows_per_tile = n_groups // NS

    src = edge_index[0].astype(jnp.int32)
    srcA = (2 * src).reshape(n_groups, G)
    srcB = (2 * src + 1).reshape(n_groups, G)
    dst = edge_index[1].astype(jnp.int32).reshape(n_groups, G)
    et = jnp.clip(edge_type, 0, T1 - 1).astype(jnp.int32).reshape(n_groups, G)
    h2 = h.reshape(2 * N, D // 2)
    zacc = jnp.zeros((N, D // 2), jnp.float32)
    zcnt = jnp.zeros((N, 16), jnp.float32)

    sc_fn = _make_sc_edge_accum(N, D, G, rows_per_tile)
    acc2, cnt = sc_fn(h2, srcA, srcB, dst, et, zacc, zcnt)

    e_pad = jnp.zeros((16, D), jnp.float32).at[:T1].set(edge_embed)
    nn = jnp.asarray(num_nodes, jnp.int32).reshape(1)
    tc_fn = _make_tc_gru(N, D, 1000, T1)
    return tc_fn(nn, acc2, cnt, h, W_msg, e_pad, b_msg.reshape(1, D),
                 W_ih, W_hh, b_ih.reshape(1, 3 * D), b_hh.reshape(1, 3 * D))


# speedup vs baseline: 11.9235x; 1.1286x over previous
import functools

import jax
import jax.numpy as jnp
from jax import lax
from jax.experimental import pallas as pl
from jax.experimental.pallas import tpu as pltpu
from jax.experimental.pallas import tpu_sc as plsc


def _make_sc_edge_accum(N, D, G, rows_per_tile):
    info = plsc.get_sparse_core_info()
    NC, NS = info.num_cores, info.num_subcores
    Dh = D // 2
    mesh = plsc.VectorSubcoreMesh(core_axis_name="c", subcore_axis_name="s")
    nrt = (N // NS) // 8 * 8
    tail = N - nrt * NS

    @functools.partial(
        pl.kernel,
        out_type=(
            jax.ShapeDtypeStruct((NC, N, Dh), jnp.float32),
            jax.ShapeDtypeStruct((NC, N, 16), jnp.float32),
        ),
        mesh=mesh,
        scratch_types=[
            pltpu.VMEM_SHARED((N, Dh), jnp.float32),
            pltpu.VMEM_SHARED((N, 16), jnp.float32),
            pltpu.VMEM((rows_per_tile // 2, G), jnp.int32),
            pltpu.VMEM((rows_per_tile // 2, G), jnp.int32),
            pltpu.VMEM((rows_per_tile // 2, G), jnp.int32),
            pltpu.VMEM((5, G, Dh), jnp.float32),
            pltpu.VMEM((5, G, 16), jnp.float32),
            pltpu.SemaphoreType.DMA((5,)),
            pltpu.SemaphoreType.DMA((5,)),
        ],
        compiler_params=pltpu.CompilerParams(use_tc_tiling_on_sc=False),
    )
    def sc_fn(h2_hbm, srcA_hbm, srcB_hbm, dst_hbm, et_hbm, zacc_hbm, zcnt_hbm,
              acc_out, cnt_out,
              acc_sh, cnt_sh, src_v, dst_v, et_v, rows_v, oh_v, sg, ss):
        c = lax.axis_index("c")
        s = lax.axis_index("s")

        ra = s * nrt
        pltpu.sync_copy(zacc_hbm.at[pl.ds(ra, nrt)], acc_sh.at[pl.ds(ra, nrt)])

        @pl.when(s == 0)
        def _():
            if tail:
                t0 = nrt * NS
                pltpu.sync_copy(zacc_hbm.at[pl.ds(t0, tail)],
                                acc_sh.at[pl.ds(t0, tail)])

        pltpu.sync_copy(zcnt_hbm.at[pl.ds(ra, nrt)],
                        cnt_sh.at[pl.ds(ra, nrt)])

        @pl.when(s == 0)
        def _():
            if tail:
                t0 = nrt * NS
                pltpu.sync_copy(zcnt_hbm.at[pl.ds(t0, tail)],
                                cnt_sh.at[pl.ds(t0, tail)])

        lanes = lax.broadcasted_iota(jnp.int32, (16,), 0)
        n = rows_per_tile
        NB = 5
        CH = n // 2

        def drain_acc(buf):
            pltpu.make_async_copy(rows_v.at[buf],
                                  acc_sh.at[dst_v.at[0]], ss.at[buf]).wait()

        def drain_cnt(buf):
            pltpu.make_async_copy(oh_v.at[buf],
                                  cnt_sh.at[dst_v.at[0]], ss.at[buf]).wait()

        def chunk_body(ci, carry):
            @pl.when(ci > 0)
            def _():
                for j in range(NB - 1):
                    gl = CH - (NB - 1) + j
                    drain_acc(gl % NB)

                    @pl.when(lax.rem(ci - 1 + gl, 2) == c)
                    def _():
                        drain_cnt(gl % NB)

            i0 = s * n + ci * CH

            @pl.when(c == 0)
            def _():
                pltpu.sync_copy(srcA_hbm.at[pl.ds(i0, CH)], src_v)

            @pl.when(c == 1)
            def _():
                pltpu.sync_copy(srcB_hbm.at[pl.ds(i0, CH)], src_v)

            pltpu.sync_copy(dst_hbm.at[pl.ds(i0, CH)], dst_v)
            pltpu.sync_copy(et_hbm.at[pl.ds(i0, CH)], et_v)

            pltpu.async_copy(h2_hbm.at[src_v.at[0]], rows_v.at[0], sg.at[0])

            def body(gl, carry2):
                b = lax.rem(gl, NB)
                nb = lax.rem(gl + 1, NB)
                par = lax.rem(ci + gl, 2) == c

                @pl.when(gl >= NB - 1)
                def _():
                    drain_acc(nb)

                @pl.when(jnp.logical_and(gl >= NB - 1, par))
                def _():
                    drain_cnt(nb)

                @pl.when(gl + 1 < CH)
                def _():
                    pltpu.async_copy(h2_hbm.at[src_v.at[gl + 1]],
                                     rows_v.at[nb], sg.at[nb])

                @pl.when(par)
                def _():
                    for jj in range(G // 16):
                        etv = et_v[gl, pl.ds(16 * jj, 16)]
                        for i in range(16):
                            oh_v[b, 16 * jj + i] = jnp.where(
                                lanes == etv[i], 1.0, 0.0)

                pltpu.make_async_copy(h2_hbm.at[src_v.at[gl]],
                                      rows_v.at[b], sg.at[b]).wait()
                pltpu.async_copy(rows_v.at[b], acc_sh.at[dst_v.at[gl]],
                                 ss.at[b], add=True)

                @pl.when(par)
                def _():
                    pltpu.async_copy(oh_v.at[b], cnt_sh.at[dst_v.at[gl]],
                                     ss.at[b], add=True)

                return carry2

            lax.fori_loop(0, CH, body, 0)
            return carry

        plsc.subcore_barrier()
        lax.fori_loop(0, n // CH, chunk_body, 0)

        for j in range(NB - 1):
            gl = CH - (NB - 1) + j
            g = n - (NB - 1) + j
            drain_acc(gl % NB)

            @pl.when((g % 2) == c)
            def _():
                drain_cnt(gl % NB)

        plsc.subcore_barrier()
        pltpu.sync_copy(acc_sh.at[pl.ds(ra, nrt)],
                        acc_out.at[c, pl.ds(ra, nrt)])

        @pl.when(s == 0)
        def _():
            if tail:
                t0 = nrt * NS
                pltpu.sync_copy(acc_sh.at[pl.ds(t0, tail)],
                                acc_out.at[c, pl.ds(t0, tail)])

        pltpu.sync_copy(cnt_sh.at[pl.ds(ra, nrt)],
                        cnt_out.at[c, pl.ds(ra, nrt)])

        @pl.when(s == 0)
        def _():
            if tail:
                t0 = nrt * NS
                pltpu.sync_copy(cnt_sh.at[pl.ds(t0, tail)],
                                cnt_out.at[c, pl.ds(t0, tail)])

    return sc_fn


def _make_tc_gru(N, D, R, T1):
    T3 = 3 * D
    Dh = D // 2
    hi = lax.Precision.DEFAULT

    def body(nn_ref, acc_ref, cnt_ref, h_ref, wm_ref, epad_ref, bmsg_ref,
             wih_ref, whh_ref, bih_ref, bhh_ref, out_ref):
        hh = h_ref[...]
        trow = lax.broadcasted_iota(jnp.int32, (16, D), 0)
        embp = epad_ref[...] + jnp.where(trow < T1, bmsg_ref[...], 0.0)
        dn = (((1,), (1,)), ((), ()))
        msgs = (
            lax.dot_general(acc_ref[0], wm_ref[:, :Dh], dn, precision=hi,
                            preferred_element_type=jnp.float32)
            + lax.dot_general(acc_ref[1], wm_ref[:, Dh:], dn, precision=hi,
                              preferred_element_type=jnp.float32)
            + jnp.dot(cnt_ref[0] + cnt_ref[1], embp, precision=hi,
                      preferred_element_type=jnp.float32)
        )
        gi = lax.dot_general(msgs, wih_ref[...], dn, precision=hi,
                             preferred_element_type=jnp.float32) + bih_ref[...]
        gh = lax.dot_general(hh, whh_ref[...], dn, precision=hi,
                             preferred_element_type=jnp.float32) + bhh_ref[...]
        r = jax.nn.sigmoid(gi[:, :D] + gh[:, :D])
        z = jax.nn.sigmoid(gi[:, D:2 * D] + gh[:, D:2 * D])
        n = jnp.tanh(gi[:, 2 * D:] + r * gh[:, 2 * D:])
        h_new = (1.0 - z) * n + z * hh
        row0 = pl.program_id(0) * R
        rows = row0 + lax.broadcasted_iota(jnp.int32, (R, D), 0)
        out_ref[...] = jnp.where(rows < nn_ref[0], h_new, 0.0)

    return pl.pallas_call(
        body,
        grid=(N // R,),
        in_specs=[
            pl.BlockSpec(memory_space=pltpu.MemorySpace.SMEM),
            pl.BlockSpec((2, R, Dh), lambda j: (0, j, 0)),
            pl.BlockSpec((2, R, 16), lambda j: (0, j, 0)),
            pl.BlockSpec((R, D), lambda j: (j, 0)),
            pl.BlockSpec((D, D), lambda j: (0, 0)),
            pl.BlockSpec((16, D), lambda j: (0, 0)),
            pl.BlockSpec((1, D), lambda j: (0, 0)),
            pl.BlockSpec((T3, D), lambda j: (0, 0)),
            pl.BlockSpec((T3, D), lambda j: (0, 0)),
            pl.BlockSpec((1, T3), lambda j: (0, 0)),
            pl.BlockSpec((1, T3), lambda j: (0, 0)),
        ],
        out_specs=pl.BlockSpec((R, D), lambda j: (j, 0)),
        out_shape=jax.ShapeDtypeStruct((N, D), jnp.float32),
    )


def kernel(h, edge_index, edge_type, num_nodes, W_msg, b_msg, edge_embed,
           W_ih, W_hh, b_ih, b_hh):
    N, D = h.shape
    E = edge_index.shape[1]
    T1 = edge_embed.shape[0]

    G = 80
    n_groups = E // G
    NS = 16
    rows_per_tile = n_groups // NS

    src = edge_index[0].astype(jnp.int32)
    srcA = (2 * src).reshape(n_groups, G)
    srcB = (2 * src + 1).reshape(n_groups, G)
    dst = edge_index[1].astype(jnp.int32).reshape(n_groups, G)
    et = jnp.clip(edge_type, 0, T1 - 1).astype(jnp.int32).reshape(n_groups, G)
    h2 = h.reshape(2 * N, D // 2)
    zacc = jnp.zeros((N, D // 2), jnp.float32)
    zcnt = jnp.zeros((N, 16), jnp.float32)

    sc_fn = _make_sc_edge_accum(N, D, G, rows_per_tile)
    acc2, cnt = sc_fn(h2, srcA, srcB, dst, et, zacc, zcnt)

    e_pad = jnp.zeros((16, D), jnp.float32).at[:T1].set(edge_embed)
    nn = jnp.asarray(num_nodes, jnp.int32).reshape(1)
    tc_fn = _make_tc_gru(N, D, 1000, T1)
    return tc_fn(nn, acc2, cnt, h, W_msg, e_pad, b_msg.reshape(1, D),
                 W_ih, W_hh, b_ih.reshape(1, 3 * D), b_hh.reshape(1, 3 * D))
